# baseline (speedup 1.0000x reference)
import functools

import jax
import jax.numpy as jnp
from jax import lax
from jax.experimental import pallas as pl
from jax.experimental.pallas import tpu as pltpu
from jax.experimental.pallas import tpu_sc as plsc

_J = 200
_B = 4096
_E = 32
_V = 1000000
_CB = 512
_LANES = 16


@functools.lru_cache(maxsize=None)
def _make_kernel():
    info = plsc.get_sparse_core_info()
    NC, NS = info.num_cores, info.num_subcores
    NW = NC * NS
    n_units = _J * (_B // _CB)
    units_per_w = n_units // NW
    qpj = _B // _CB
    out_words = _J * _E * _B
    mesh = plsc.VectorSubcoreMesh(core_axis_name="c", subcore_axis_name="s")

    @functools.partial(
        pl.kernel,
        mesh=mesh,
        out_type=jax.ShapeDtypeStruct((out_words,), jnp.float32),
        scratch_types=[
            pltpu.VMEM((_CB,), jnp.int32),
            pltpu.VMEM((_CB, _E), jnp.float32),
            pltpu.VMEM((_E * _CB,), jnp.float32),
            pltpu.SemaphoreType.DMA,
        ],
        compiler_params=pltpu.CompilerParams(
            use_tc_tiling_on_sc=False, needs_layout_passes=False
        ),
    )
    def k(idx_hbm, table_hbm, out_hbm, idx_v, rows_v, dst_v, sem):
        wid = lax.axis_index("s") * NC + lax.axis_index("c")
        lanes = lax.iota(jnp.int32, _LANES)
        base0 = (lanes >> 3) * (8 * _CB) + (lanes & 7) * 128
        base1 = base0 + 2 * (8 * _CB)

        def unit_body(t, carry):
            u = wid + t * NW
            j = u // qpj
            q = u % qpj
            src = pl.multiple_of(j * _B + q * _CB, 8)
            pltpu.sync_copy(idx_hbm.at[pl.ds(src, _CB)], idx_v)
            pltpu.async_copy(table_hbm.at[idx_v], rows_v, sem).wait()

            def b_body(b, carry2):
                sb = (b >> 7) * 1024 + (b & 127)
                sbv = jnp.broadcast_to(sb, (_LANES,))
                v0 = rows_v[b, pl.ds(0, _LANES)]
                v1 = rows_v[b, pl.ds(_LANES, _LANES)]
                plsc.store_scatter(dst_v, [base0 + sbv], v0)
                plsc.store_scatter(dst_v, [base1 + sbv], v1)
                return carry2

            lax.fori_loop(0, _CB, b_body, 0)

            base = j * (_E * _B) + q * (_CB * 8)
            for e_hi in range(_E // 8):
                dsto = pl.multiple_of(base + e_hi * (8 * _B), 8)
                pltpu.sync_copy(
                    dst_v.at[pl.ds(e_hi * 8 * _CB, 8 * _CB)],
                    out_hbm.at[pl.ds(dsto, 8 * _CB)],
                )
            return carry

        lax.fori_loop(0, units_per_w, unit_body, 0)

    return k


def kernel(x, seg_emb_weight):
    xflat = jnp.swapaxes(x, 0, 1).reshape(_J * _B).astype(jnp.int32)
    tbl128 = lax.optimization_barrier(seg_emb_weight.reshape(_V // 4, 128))
    tbl = tbl128.reshape(_V, _E)
    flat = _make_kernel()(xflat, tbl)
    out5 = flat.reshape(_J, _E // 8, _B // 128, 8, 128)
    return out5.transpose(2, 4, 0, 1, 3).reshape(_B, _J, _E)


# speedup vs baseline: 1.0001x; 1.0001x over previous
import functools

import jax
import jax.numpy as jnp
from jax import lax
from jax.experimental import pallas as pl
from jax.experimental.pallas import tpu as pltpu
from jax.experimental.pallas import tpu_sc as plsc

_J = 200
_B = 4096
_E = 32
_V = 1000000
_CB = 512
_LANES = 16


@functools.lru_cache(maxsize=None)
def _make_kernel():
    info = plsc.get_sparse_core_info()
    NC, NS = info.num_cores, info.num_subcores
    NW = NC * NS
    n_units = _J * (_B // _CB)
    units_per_w = n_units // NW
    qpj = _B // _CB
    out_words = _J * _E * _B
    mesh = plsc.VectorSubcoreMesh(core_axis_name="c", subcore_axis_name="s")

    @functools.partial(
        pl.kernel,
        mesh=mesh,
        out_type=jax.ShapeDtypeStruct((out_words,), jnp.float32),
        scratch_types=[
            pltpu.VMEM((_CB,), jnp.int32),
            pltpu.VMEM((_CB, _E), jnp.float32),
            pltpu.VMEM((_E * _CB,), jnp.float32),
            pltpu.SemaphoreType.DMA,
        ],
        compiler_params=pltpu.CompilerParams(
            use_tc_tiling_on_sc=False, needs_layout_passes=False
        ),
    )
    def k(idx_hbm, table_hbm, out_hbm, idx_v, rows_v, dst_v, sem):
        wid = lax.axis_index("s") * NC + lax.axis_index("c")
        lanes = lax.iota(jnp.int32, _LANES)
        base0 = (lanes >> 3) * (8 * _CB) + (lanes & 7) * 128
        base1 = base0 + 2 * (8 * _CB)
        bases = [(base0 + u, base1 + u) for u in range(8)]

        def unit_body(t, carry):
            u = wid + t * NW
            j = u // qpj
            q = u % qpj
            src = pl.multiple_of(j * _B + q * _CB, 8)
            pltpu.sync_copy(idx_hbm.at[pl.ds(src, _CB)], idx_v)
            pltpu.async_copy(table_hbm.at[idx_v], rows_v, sem).wait()

            def g_body(g, carry2):
                b0 = g * 8
                sb = (b0 >> 7) * 1024 + (b0 & 127)
                sbv = jnp.broadcast_to(sb, (_LANES,))
                for u in range(8):
                    b = b0 + u
                    v0 = rows_v[b, pl.ds(0, _LANES)]
                    v1 = rows_v[b, pl.ds(_LANES, _LANES)]
                    plsc.store_scatter(dst_v, [bases[u][0] + sbv], v0)
                    plsc.store_scatter(dst_v, [bases[u][1] + sbv], v1)
                return carry2

            lax.fori_loop(0, _CB // 8, g_body, 0)

            base = j * (_E * _B) + q * (_CB * 8)
            for e_hi in range(_E // 8):
                dsto = pl.multiple_of(base + e_hi * (8 * _B), 8)
                pltpu.sync_copy(
                    dst_v.at[pl.ds(e_hi * 8 * _CB, 8 * _CB)],
                    out_hbm.at[pl.ds(dsto, 8 * _CB)],
                )
            return carry

        lax.fori_loop(0, units_per_w, unit_body, 0)

    return k


def kernel(x, seg_emb_weight):
    xflat = jnp.swapaxes(x, 0, 1).reshape(_J * _B).astype(jnp.int32)
    tbl128 = lax.optimization_barrier(seg_emb_weight.reshape(_V // 4, 128))
    tbl = tbl128.reshape(_V, _E)
    flat = _make_kernel()(xflat, tbl)
    out5 = flat.reshape(_J, _E // 8, _B // 128, 8, 128)
    return out5.transpose(2, 4, 0, 1, 3).reshape(_B, _J, _E)


# speedup vs baseline: 1.1360x; 1.1359x over previous
import functools

import jax
import jax.numpy as jnp
from jax import lax
from jax.experimental import pallas as pl
from jax.experimental.pallas import tpu as pltpu
from jax.experimental.pallas import tpu_sc as plsc

_J = 200
_B = 4096
_E = 32
_V = 1000000
_CB = 512
_LANES = 16


@functools.lru_cache(maxsize=None)
def _make_kernel():
    info = plsc.get_sparse_core_info()
    NC, NS = info.num_cores, info.num_subcores
    NW = NC * NS
    n_units = _J * (_B // _CB)
    units_per_w = n_units // NW
    qpj = _B // _CB
    out_words = _J * _E * _B
    mesh = plsc.VectorSubcoreMesh(core_axis_name="c", subcore_axis_name="s")

    @functools.partial(
        pl.kernel,
        mesh=mesh,
        out_type=jax.ShapeDtypeStruct((out_words,), jnp.float32),
        scratch_types=[
            pltpu.VMEM((2, _CB), jnp.int32),
            pltpu.VMEM((2, _CB, _E), jnp.float32),
            pltpu.VMEM((2, _E * _CB), jnp.float32),
            pltpu.SemaphoreType.DMA,
            pltpu.SemaphoreType.DMA,
            pltpu.SemaphoreType.DMA,
            pltpu.SemaphoreType.DMA,
            pltpu.SemaphoreType.DMA,
            pltpu.SemaphoreType.DMA,
        ],
        compiler_params=pltpu.CompilerParams(
            use_tc_tiling_on_sc=False, needs_layout_passes=False
        ),
    )
    def k(idx_hbm, table_hbm, out_hbm, idx_v, rows_v, dst_v,
          si0, si1, sg0, sg1, so0, so1):
        wid = lax.axis_index("s") * NC + lax.axis_index("c")
        lanes = lax.iota(jnp.int32, _LANES)
        base0 = (lanes >> 3) * (8 * _CB) + (lanes & 7) * 128
        base1 = base0 + 2 * (8 * _CB)
        bases = [(base0 + u, base1 + u) for u in range(8)]
        si = (si0, si1)
        sg = (sg0, sg1)
        so = (so0, so1)

        def idx_src(t):
            u = wid + t * NW
            j = u // qpj
            q = u % qpj
            return idx_hbm.at[pl.ds(pl.multiple_of(j * _B + q * _CB, 8), _CB)]

        def out_dst(t, e_hi):
            u = wid + t * NW
            j = u // qpj
            q = u % qpj
            o = pl.multiple_of(j * (_E * _B) + q * (_CB * 8) + e_hi * (8 * _B), 8)
            return out_hbm.at[pl.ds(o, 8 * _CB)]

        def shuffle(r):
            rows = rows_v.at[r]
            dst = dst_v.at[r]

            def g_body(g, carry2):
                b0 = g * 8
                sb = (b0 >> 7) * 1024 + (b0 & 127)
                sbv = jnp.broadcast_to(sb, (_LANES,))
                for u in range(8):
                    b = b0 + u
                    v0 = rows[b, pl.ds(0, _LANES)]
                    v1 = rows[b, pl.ds(_LANES, _LANES)]
                    plsc.store_scatter(dst, [bases[u][0] + sbv], v0)
                    plsc.store_scatter(dst, [bases[u][1] + sbv], v1)
                return carry2

            lax.fori_loop(0, _CB // 8, g_body, 0)

        pltpu.async_copy(idx_src(0), idx_v.at[0], si0).wait()
        pltpu.async_copy(table_hbm.at[idx_v.at[0]], rows_v.at[0], sg0)
        pltpu.async_copy(idx_src(1), idx_v.at[1], si1)

        def tt_body(tt, carry):
            for r in range(2):
                t = 2 * tt + r
                pltpu.make_async_copy(
                    table_hbm.at[idx_v.at[r]], rows_v.at[r], sg[r]
                ).wait()

                @pl.when(t + 1 < units_per_w)
                def _():
                    pltpu.make_async_copy(
                        idx_src(t + 1), idx_v.at[1 - r], si[1 - r]
                    ).wait()
                    pltpu.async_copy(
                        table_hbm.at[idx_v.at[1 - r]], rows_v.at[1 - r], sg[1 - r]
                    )

                @pl.when(t + 2 < units_per_w)
                def _():
                    pltpu.async_copy(idx_src(t + 2), idx_v.at[r], si[r])

                @pl.when(t >= 2)
                def _():
                    for e_hi in range(_E // 8):
                        pltpu.make_async_copy(
                            dst_v.at[r].at[pl.ds(e_hi * 8 * _CB, 8 * _CB)],
                            out_dst(t - 2, e_hi),
                            so[r],
                        ).wait()

                shuffle(r)
                for e_hi in range(_E // 8):
                    pltpu.async_copy(
                        dst_v.at[r].at[pl.ds(e_hi * 8 * _CB, 8 * _CB)],
                        out_dst(t, e_hi),
                        so[r],
                    )
            return carry

        lax.fori_loop(0, units_per_w // 2, tt_body, 0)

        for r in range(2):
            t = units_per_w - 2 + r
            for e_hi in range(_E // 8):
                pltpu.make_async_copy(
                    dst_v.at[r].at[pl.ds(e_hi * 8 * _CB, 8 * _CB)],
                    out_dst(t, e_hi),
                    so[r],
                ).wait()

    return k


def kernel(x, seg_emb_weight):
    xflat = jnp.swapaxes(x, 0, 1).reshape(_J * _B).astype(jnp.int32)
    tbl128 = lax.optimization_barrier(seg_emb_weight.reshape(_V // 4, 128))
    tbl = tbl128.reshape(_V, _E)
    flat = _make_kernel()(xflat, tbl)
    out5 = flat.reshape(_J, _E // 8, _B // 128, 8, 128)
    return out5.transpose(2, 4, 0, 1, 3).reshape(_B, _J, _E)


# speedup vs baseline: 1.1366x; 1.0005x over previous
import functools

import jax
import jax.numpy as jnp
from jax import lax
from jax.experimental import pallas as pl
from jax.experimental.pallas import tpu as pltpu
from jax.experimental.pallas import tpu_sc as plsc

_J = 200
_B = 4096
_E = 32
_V = 1000000
_CB = 512
_LANES = 16


@functools.lru_cache(maxsize=None)
def _make_kernel():
    info = plsc.get_sparse_core_info()
    NC, NS = info.num_cores, info.num_subcores
    NW = NC * NS
    n_units = _J * (_B // _CB)
    units_per_w = n_units // NW
    qpj = _B // _CB
    out_words = _J * _E * _B
    mesh = plsc.VectorSubcoreMesh(core_axis_name="c", subcore_axis_name="s")

    @functools.partial(
        pl.kernel,
        mesh=mesh,
        out_type=jax.ShapeDtypeStruct((out_words,), jnp.float32),
        scratch_types=[
            pltpu.VMEM((_CB,), jnp.int32),
            pltpu.VMEM((_CB,), jnp.int32),
            pltpu.VMEM((_CB, _E), jnp.float32),
            pltpu.VMEM((_CB, _E), jnp.float32),
            pltpu.VMEM((_E * _CB,), jnp.float32),
            pltpu.VMEM((_E * _CB,), jnp.float32),
            pltpu.SemaphoreType.DMA,
            pltpu.SemaphoreType.DMA,
            pltpu.SemaphoreType.DMA,
            pltpu.SemaphoreType.DMA,
            pltpu.SemaphoreType.DMA,
            pltpu.SemaphoreType.DMA,
        ],
        compiler_params=pltpu.CompilerParams(
            use_tc_tiling_on_sc=False, needs_layout_passes=False
        ),
    )
    def k(idx_hbm, table_hbm, out_hbm, idx_v0, idx_v1, rows_v0, rows_v1,
          dst_v0, dst_v1, si0, si1, sg0, sg1, so0, so1):
        idx_v = (idx_v0, idx_v1)
        rows_v = (rows_v0, rows_v1)
        dst_v = (dst_v0, dst_v1)
        wid = lax.axis_index("s") * NC + lax.axis_index("c")
        lanes = lax.iota(jnp.int32, _LANES)
        base0 = (lanes >> 3) * (8 * _CB) + (lanes & 7) * 128
        base1 = base0 + 2 * (8 * _CB)
        bases = [(base0 + u, base1 + u) for u in range(8)]
        si = (si0, si1)
        sg = (sg0, sg1)
        so = (so0, so1)

        def idx_src(t):
            u = wid + t * NW
            j = u // qpj
            q = u % qpj
            return idx_hbm.at[pl.ds(pl.multiple_of(j * _B + q * _CB, 8), _CB)]

        def out_dst(t, e_hi):
            u = wid + t * NW
            j = u // qpj
            q = u % qpj
            o = pl.multiple_of(j * (_E * _B) + q * (_CB * 8) + e_hi * (8 * _B), 8)
            return out_hbm.at[pl.ds(o, 8 * _CB)]

        def shuffle(r):
            rows = rows_v[r]
            dst = dst_v[r]

            def g_body(g, carry2):
                b0 = g * 8
                sb = (b0 >> 7) * 1024 + (b0 & 127)
                sbv = jnp.broadcast_to(sb, (_LANES,))
                for u in range(8):
                    b = b0 + u
                    v0 = rows[b, pl.ds(0, _LANES)]
                    v1 = rows[b, pl.ds(_LANES, _LANES)]
                    plsc.store_scatter(dst, [bases[u][0] + sbv], v0)
                    plsc.store_scatter(dst, [bases[u][1] + sbv], v1)
                return carry2

            lax.fori_loop(0, _CB // 8, g_body, 0)

        pltpu.async_copy(idx_src(0), idx_v[0], si0).wait()
        pltpu.async_copy(table_hbm.at[idx_v[0]], rows_v[0], sg0)
        pltpu.async_copy(idx_src(1), idx_v[1], si1)

        def tt_body(tt, carry):
            for r in range(2):
                t = 2 * tt + r
                pltpu.make_async_copy(
                    table_hbm.at[idx_v[r]], rows_v[r], sg[r]
                ).wait()

                @pl.when(t + 1 < units_per_w)
                def _():
                    pltpu.make_async_copy(
                        idx_src(t + 1), idx_v[1 - r], si[1 - r]
                    ).wait()
                    pltpu.async_copy(
                        table_hbm.at[idx_v[1 - r]], rows_v[1 - r], sg[1 - r]
                    )

                @pl.when(t + 2 < units_per_w)
                def _():
                    pltpu.async_copy(idx_src(t + 2), idx_v[r], si[r])

                @pl.when(t >= 2)
                def _():
                    for e_hi in range(_E // 8):
                        pltpu.make_async_copy(
                            dst_v[r].at[pl.ds(e_hi * 8 * _CB, 8 * _CB)],
                            out_dst(t - 2, e_hi),
                            so[r],
                        ).wait()

                shuffle(r)
                for e_hi in range(_E // 8):
                    pltpu.async_copy(
                        dst_v[r].at[pl.ds(e_hi * 8 * _CB, 8 * _CB)],
                        out_dst(t, e_hi),
                        so[r],
                    )
            return carry

        lax.fori_loop(0, units_per_w // 2, tt_body, 0)

        for r in range(2):
            t = units_per_w - 2 + r
            for e_hi in range(_E // 8):
                pltpu.make_async_copy(
                    dst_v[r].at[pl.ds(e_hi * 8 * _CB, 8 * _CB)],
                    out_dst(t, e_hi),
                    so[r],
                ).wait()

    return k


def kernel(x, seg_emb_weight):
    xflat = jnp.swapaxes(x, 0, 1).reshape(_J * _B).astype(jnp.int32)
    tbl128 = lax.optimization_barrier(seg_emb_weight.reshape(_V // 4, 128))
    tbl = tbl128.reshape(_V, _E)
    flat = _make_kernel()(xflat, tbl)
    out5 = flat.reshape(_J, _E // 8, _B // 128, 8, 128)
    return out5.transpose(2, 4, 0, 1, 3).reshape(_B, _J, _E)


# speedup vs baseline: 1.7458x; 1.5360x over previous
import functools

import jax
import jax.numpy as jnp
from jax import lax
from jax.experimental import pallas as pl
from jax.experimental.pallas import tpu as pltpu
from jax.experimental.pallas import tpu_sc as plsc

_J = 200
_B = 4096
_E = 32
_V = 1000000
_CB = 512
_LANES = 16


@functools.lru_cache(maxsize=None)
def _make_kernel():
    info = plsc.get_sparse_core_info()
    NC, NS = info.num_cores, info.num_subcores
    NW = NC * NS
    n_units = _J * (_B // _CB)
    units_per_w = n_units // NW
    qpj = _B // _CB
    out_words = _J * _E * _B
    mesh = plsc.VectorSubcoreMesh(core_axis_name="c", subcore_axis_name="s")

    @functools.partial(
        pl.kernel,
        mesh=mesh,
        out_type=jax.ShapeDtypeStruct((out_words // 128, 128), jnp.float32),
        scratch_types=[
            pltpu.VMEM((_CB,), jnp.int32),
            pltpu.VMEM((_CB,), jnp.int32),
            pltpu.VMEM((_CB, _E), jnp.float32),
            pltpu.VMEM((_CB, _E), jnp.float32),
            pltpu.VMEM((160, 129), jnp.float32),
            pltpu.VMEM((160, 129), jnp.float32),
            pltpu.SemaphoreType.DMA,
            pltpu.SemaphoreType.DMA,
            pltpu.SemaphoreType.DMA,
            pltpu.SemaphoreType.DMA,
            pltpu.SemaphoreType.DMA,
            pltpu.SemaphoreType.DMA,
        ],
        compiler_params=pltpu.CompilerParams(
            use_tc_tiling_on_sc=False, needs_layout_passes=False
        ),
    )
    def k(idx_hbm, table_hbm, out_hbm, idx_v0, idx_v1, rows_v0, rows_v1,
          dst_v0, dst_v1, si0, si1, sg0, sg1, so0, so1):
        idx_v = (idx_v0, idx_v1)
        rows_v = (rows_v0, rows_v1)
        dst_v = (dst_v0, dst_v1)
        wid = lax.axis_index("s") * NC + lax.axis_index("c")
        lanes = lax.iota(jnp.int32, _LANES)
        row0 = (lanes >> 3) * 40 + (lanes & 7)
        row1 = row0 + 2 * 40
        si = (si0, si1)
        sg = (sg0, sg1)
        so = (so0, so1)

        def idx_src(t):
            u = wid + t * NW
            j = u // qpj
            q = u % qpj
            return idx_hbm.at[pl.ds(pl.multiple_of(j * _B + q * _CB, 8), _CB)]

        def out_dst(t, e_hi):
            u = wid + t * NW
            j = u // qpj
            q = u % qpj
            row = pl.multiple_of(j * 1024 + e_hi * 256 + q * 32, 8)
            return out_hbm.at[pl.ds(row, 32), :]

        def shuffle(r):
            rows = rows_v[r]
            dst = dst_v[r]

            def g_body(g, carry2):
                b0 = g * 8
                rbv = jnp.broadcast_to((b0 >> 7) * 8, (_LANES,))
                rv0 = row0 + rbv
                rv1 = row1 + rbv
                cb = b0 & 127
                for u in range(8):
                    b = b0 + u
                    cv = jnp.broadcast_to(cb + u, (_LANES,))
                    v0 = rows[b, pl.ds(0, _LANES)]
                    v1 = rows[b, pl.ds(_LANES, _LANES)]
                    plsc.store_scatter(dst, [rv0, cv], v0)
                    plsc.store_scatter(dst, [rv1, cv], v1)
                return carry2

            lax.fori_loop(0, _CB // 8, g_body, 0)

        pltpu.async_copy(idx_src(0), idx_v[0], si0).wait()
        pltpu.async_copy(table_hbm.at[idx_v[0]], rows_v[0], sg0)
        pltpu.async_copy(idx_src(1), idx_v[1], si1)

        def tt_body(tt, carry):
            for r in range(2):
                t = 2 * tt + r
                pltpu.make_async_copy(
                    table_hbm.at[idx_v[r]], rows_v[r], sg[r]
                ).wait()

                @pl.when(t + 1 < units_per_w)
                def _():
                    pltpu.make_async_copy(
                        idx_src(t + 1), idx_v[1 - r], si[1 - r]
                    ).wait()
                    pltpu.async_copy(
                        table_hbm.at[idx_v[1 - r]], rows_v[1 - r], sg[1 - r]
                    )

                @pl.when(t + 2 < units_per_w)
                def _():
                    pltpu.async_copy(idx_src(t + 2), idx_v[r], si[r])

                @pl.when(t >= 2)
                def _():
                    for e_hi in range(_E // 8):
                        pltpu.make_async_copy(
                            dst_v[r].at[pl.ds(e_hi * 40, 32), pl.ds(0, 128)],
                            out_dst(t - 2, e_hi),
                            so[r],
                        ).wait()

                shuffle(r)
                for e_hi in range(_E // 8):
                    pltpu.async_copy(
                        dst_v[r].at[pl.ds(e_hi * 40, 32), pl.ds(0, 128)],
                        out_dst(t, e_hi),
                        so[r],
                    )
            return carry

        lax.fori_loop(0, units_per_w // 2, tt_body, 0)

        for r in range(2):
            t = units_per_w - 2 + r
            for e_hi in range(_E // 8):
                pltpu.make_async_copy(
                    dst_v[r].at[pl.ds(e_hi * 40, 32), pl.ds(0, 128)],
                    out_dst(t, e_hi),
                    so[r],
                ).wait()

    return k


def kernel(x, seg_emb_weight):
    xflat = jnp.swapaxes(x, 0, 1).reshape(_J * _B).astype(jnp.int32)
    tbl128 = lax.optimization_barrier(seg_emb_weight.reshape(_V // 4, 128))
    tbl = tbl128.reshape(_V, _E)
    out2 = _make_kernel()(xflat, tbl)
    out5 = out2.reshape(_J, _E // 8, _B // 128, 8, 128)
    return out5.transpose(2, 4, 0, 1, 3).reshape(_B, _J, _E)


# speedup vs baseline: 1.8766x; 1.0749x over previous
import functools

import jax
import jax.numpy as jnp
from jax import lax
from jax.experimental import pallas as pl
from jax.experimental.pallas import tpu as pltpu
from jax.experimental.pallas import tpu_sc as plsc

_J = 200
_B = 4096
_E = 32
_V = 1000000
_CB = 512
_LANES = 16


@functools.lru_cache(maxsize=None)
def _make_kernel():
    info = plsc.get_sparse_core_info()
    NC, NS = info.num_cores, info.num_subcores
    NW = NC * NS
    n_units = _J * (_B // _CB)
    units_per_w = n_units // NW
    qpj = _B // _CB
    out_words = _J * _E * _B
    mesh = plsc.VectorSubcoreMesh(core_axis_name="c", subcore_axis_name="s")

    @functools.partial(
        pl.kernel,
        mesh=mesh,
        out_type=jax.ShapeDtypeStruct((out_words // 128, 128), jnp.float32),
        scratch_types=[
            pltpu.VMEM((_CB,), jnp.int32),
            pltpu.VMEM((_CB,), jnp.int32),
            pltpu.VMEM((_CB, _E), jnp.float32),
            pltpu.VMEM((_CB, _E), jnp.float32),
            pltpu.VMEM((160, 129), jnp.float32),
            pltpu.VMEM((160, 129), jnp.float32),
            pltpu.SemaphoreType.DMA,
            pltpu.SemaphoreType.DMA,
            pltpu.SemaphoreType.DMA,
            pltpu.SemaphoreType.DMA,
            pltpu.SemaphoreType.DMA,
            pltpu.SemaphoreType.DMA,
        ],
        compiler_params=pltpu.CompilerParams(
            use_tc_tiling_on_sc=False, needs_layout_passes=False
        ),
    )
    def k(idx_hbm, table_hbm, out_hbm, idx_v0, idx_v1, rows_v0, rows_v1,
          dst_v0, dst_v1, si0, si1, sg0, sg1, so0, so1):
        idx_v = (idx_v0, idx_v1)
        rows_v = (rows_v0, rows_v1)
        dst_v = (dst_v0, dst_v1)
        wid = lax.axis_index("s") * NC + lax.axis_index("c")
        lanes = lax.iota(jnp.int32, _LANES)
        row0 = (lanes >> 3) * 40 + (lanes & 7)
        row1 = row0 + 2 * 40
        si = (si0, si1)
        sg = (sg0, sg1)
        so = (so0, so1)

        def idx_src(t):
            u = wid + t * NW
            j = u // qpj
            q = u % qpj
            return idx_hbm.at[pl.ds(pl.multiple_of(j * _B + q * _CB, 8), _CB)]

        def out_dst(t, e_hi):
            u = wid + t * NW
            j = u // qpj
            q = u % qpj
            row = pl.multiple_of(j * 1024 + e_hi * 256 + q * 32, 8)
            return out_hbm.at[pl.ds(row, 32), :]

        def shuffle(r):
            rows = rows_v[r]
            dst = dst_v[r]

            def g_body(g, carry2):
                b0 = g * 8
                rbv = jnp.broadcast_to((b0 >> 7) * 8, (_LANES,))
                rv0 = row0 + rbv
                rv1 = row1 + rbv
                cb = b0 & 127
                vals = []
                for u in range(8):
                    b = b0 + u
                    vals.append(rows[b, pl.ds(0, _LANES)])
                    vals.append(rows[b, pl.ds(_LANES, _LANES)])
                for u in range(8):
                    cv = jnp.broadcast_to(cb + u, (_LANES,))
                    plsc.store_scatter(dst, [rv0, cv], vals[2 * u])
                    plsc.store_scatter(dst, [rv1, cv], vals[2 * u + 1])
                return carry2

            lax.fori_loop(0, _CB // 8, g_body, 0)

        pltpu.async_copy(idx_src(0), idx_v[0], si0).wait()
        pltpu.async_copy(table_hbm.at[idx_v[0]], rows_v[0], sg0)
        pltpu.async_copy(idx_src(1), idx_v[1], si1)

        def tt_body(tt, carry):
            for r in range(2):
                t = 2 * tt + r
                pltpu.make_async_copy(
                    table_hbm.at[idx_v[r]], rows_v[r], sg[r]
                ).wait()

                @pl.when(t + 1 < units_per_w)
                def _():
                    pltpu.make_async_copy(
                        idx_src(t + 1), idx_v[1 - r], si[1 - r]
                    ).wait()
                    pltpu.async_copy(
                        table_hbm.at[idx_v[1 - r]], rows_v[1 - r], sg[1 - r]
                    )

                @pl.when(t + 2 < units_per_w)
                def _():
                    pltpu.async_copy(idx_src(t + 2), idx_v[r], si[r])

                @pl.when(t >= 2)
                def _():
                    for e_hi in range(_E // 8):
                        pltpu.make_async_copy(
                            dst_v[r].at[pl.ds(e_hi * 40, 32), pl.ds(0, 128)],
                            out_dst(t - 2, e_hi),
                            so[r],
                        ).wait()

                shuffle(r)
                for e_hi in range(_E // 8):
                    pltpu.async_copy(
                        dst_v[r].at[pl.ds(e_hi * 40, 32), pl.ds(0, 128)],
                        out_dst(t, e_hi),
                        so[r],
                    )
            return carry

        lax.fori_loop(0, units_per_w // 2, tt_body, 0)

        for r in range(2):
            t = units_per_w - 2 + r
            for e_hi in range(_E // 8):
                pltpu.make_async_copy(
                    dst_v[r].at[pl.ds(e_hi * 40, 32), pl.ds(0, 128)],
                    out_dst(t, e_hi),
                    so[r],
                ).wait()

    return k


def kernel(x, seg_emb_weight):
    xflat = jnp.swapaxes(x, 0, 1).reshape(_J * _B).astype(jnp.int32)
    tbl128 = lax.optimization_barrier(seg_emb_weight.reshape(_V // 4, 128))
    tbl = tbl128.reshape(_V, _E)
    out2 = _make_kernel()(xflat, tbl)
    out5 = out2.reshape(_J, _E // 8, _B // 128, 8, 128)
    return out5.transpose(2, 4, 0, 1, 3).reshape(_B, _J, _E)
